# baseline (speedup 1.0000x reference)
import functools

import jax
import jax.numpy as jnp
from jax import lax
from jax.experimental import pallas as pl
from jax.experimental.pallas import tpu as pltpu
from jax.experimental.pallas import tpu_sc as plsc

EMB_DIM = 64
CTX = 10
B_POS = 16384
B_NEG = 81920
B_TOT = B_POS + B_NEG
NC = 2
NS = 16
NW = NC * NS
ITEMS_PER_W = B_TOT // NW
C = 24
NCHUNK = ITEMS_PER_W // C
NBUF = 2
IDXG = 120
NGATHER = (C * CTX) // IDXG
OUT_LEN = B_TOT * 16
R128 = OUT_LEN // 128


def _sc_body(combo, all_u, all_v, out,
             idx_u_all, idx_v_all, rows_t0, rows_t1, rows_v0, rows_v1,
             parts0, parts1, semg0, semg1, semo0, semo1):
    rows_t = (rows_t0, rows_t1)
    rows_v = (rows_v0, rows_v1)
    parts = (parts0, parts1)
    semg = (semg0, semg1)
    semo = (semo0, semo1)

    wid = lax.axis_index("s") * NC + lax.axis_index("c")
    base = wid * ITEMS_PER_W
    pltpu.sync_copy(all_u.at[pl.ds(base, ITEMS_PER_W)], idx_u_all)
    pltpu.sync_copy(all_v.at[pl.ds(base * CTX, ITEMS_PER_W * CTX)], idx_v_all)

    def issue(j, b):
        pltpu.async_copy(
            combo.at[idx_u_all.at[pl.ds(j * C, C)]], rows_t[b], semg[b])
        for k in range(NGATHER):
            pltpu.async_copy(
                combo.at[idx_v_all.at[pl.ds(j * (C * CTX) + k * IDXG, IDXG)]],
                rows_v[b].at[pl.ds(k * IDXG, IDXG)], semg[b])

    def drain_gathers(b):
        pltpu.make_async_copy(combo.at[pl.ds(0, C)], rows_t[b], semg[b]).wait()
        pltpu.make_async_copy(
            combo.at[pl.ds(0, C * CTX)], rows_v[b], semg[b]).wait()

    def compute(j, b):
        rv, rt, pt = rows_v[b], rows_t[b], parts[b]

        def item_body(i, carry):
            r = i * CTX
            accs = [rv[r, pl.ds(d * 16, 16)] for d in range(4)]
            for c in range(1, CTX):
                for d in range(4):
                    accs[d] = accs[d] + rv[r + c, pl.ds(d * 16, 16)]
            part = accs[0] * rt[i, pl.ds(64, 16)]
            for d in range(1, 4):
                part = part + accs[d] * rt[i, pl.ds(64 + d * 16, 16)]
            pt[pl.ds(i * 16, 16)] = part
            return carry

        lax.fori_loop(0, C, item_body, 0)

    issue(0, 0)

    def outer(g, carry):
        for b in range(NBUF):
            j = g * NBUF + b
            jn = j + 1

            @pl.when(jn < NCHUNK)
            def _():
                issue(jn, b ^ 1)

            drain_gathers(b)

            @pl.when(j >= NBUF)
            def _():
                pltpu.make_async_copy(
                    parts[b], out.at[pl.ds(0, C * 16)], semo[b]).wait()

            compute(j, b)
            pltpu.async_copy(
                parts[b], out.at[pl.ds((base + j * C) * 16, C * 16)], semo[b])
        return carry

    lax.fori_loop(0, NCHUNK // NBUF, outer, 0)
    for b in range(NBUF):
        pltpu.make_async_copy(
            parts[b], out.at[pl.ds(0, C * 16)], semo[b]).wait()


_sc_scores = functools.partial(
    pl.kernel,
    out_type=jax.ShapeDtypeStruct((OUT_LEN,), jnp.float32),
    mesh=plsc.VectorSubcoreMesh(core_axis_name="c", subcore_axis_name="s"),
    scratch_types=[
        pltpu.VMEM((ITEMS_PER_W,), jnp.int32),
        pltpu.VMEM((ITEMS_PER_W * CTX,), jnp.int32),
        pltpu.VMEM((C, 2 * EMB_DIM), jnp.float32),
        pltpu.VMEM((C, 2 * EMB_DIM), jnp.float32),
        pltpu.VMEM((C * CTX, 2 * EMB_DIM), jnp.float32),
        pltpu.VMEM((C * CTX, 2 * EMB_DIM), jnp.float32),
        pltpu.VMEM((C * 16,), jnp.float32),
        pltpu.VMEM((C * 16,), jnp.float32),
        pltpu.SemaphoreType.DMA,
        pltpu.SemaphoreType.DMA,
        pltpu.SemaphoreType.DMA,
        pltpu.SemaphoreType.DMA,
    ],
    compiler_params=pltpu.CompilerParams(use_tc_tiling_on_sc=True),
)(_sc_body)


def _tc_body(parts_ref, o_ref):
    x = parts_ref[...]
    sel = (lax.broadcasted_iota(jnp.int32, (128, 8), 0) // 16
           == lax.broadcasted_iota(jnp.int32, (128, 8), 1)).astype(jnp.float32)
    s = jnp.dot(x, sel, preferred_element_type=jnp.float32)
    row = lax.broadcasted_iota(jnp.int32, (R128, 8), 0)
    sign = jnp.where(row < B_POS // 8, 1.0, -1.0)
    t = s * sign
    ls = jnp.minimum(t, 0.0) - jnp.log(1.0 + jnp.exp(-jnp.abs(t)))
    o_ref[0, 0] = -jnp.sum(ls)


_tc_finish = pl.pallas_call(
    _tc_body,
    out_shape=jax.ShapeDtypeStruct((1, 1), jnp.float32),
    out_specs=pl.BlockSpec(memory_space=pltpu.SMEM),
)


def kernel(pos_u, pos_v, neg_u, neg_v, u_emb, v_emb):
    combo = jnp.concatenate([u_emb, v_emb], axis=1)
    all_u = jnp.concatenate([pos_u, neg_u]).astype(jnp.int32)
    all_v = jnp.concatenate(
        [pos_v.reshape(-1), neg_v.reshape(-1)]).astype(jnp.int32)
    parts = _sc_scores(combo, all_u, all_v)
    loss = _tc_finish(parts.reshape(R128, 128))
    return loss[0, 0]


# speedup vs baseline: 1.2810x; 1.2810x over previous
import functools

import jax
import jax.numpy as jnp
from jax import lax
from jax.experimental import pallas as pl
from jax.experimental.pallas import tpu as pltpu
from jax.experimental.pallas import tpu_sc as plsc

EMB_DIM = 64
CTX = 10
B_POS = 16384
B_NEG = 81920
B_TOT = B_POS + B_NEG
TROWS = 199999
NC = 2
NS = 16
NW = NC * NS
POS_PER_W = B_POS // NW
NEG_PER_W = B_NEG // NW
C = 32
NPOS_CHUNK = POS_PER_W // C
NNEG_CHUNK = NEG_PER_W // C
NCHUNK = NPOS_CHUNK + NNEG_CHUNK
ITEMS_PER_W = POS_PER_W + NEG_PER_W
NBUF = 2
OUT_LEN = B_TOT * 16
R128 = OUT_LEN // 128
POS_CTX_LEN = POS_PER_W * CTX


def _prep_idx_body(pvt_ref, nvt_ref, opv_ref, onv_ref):
    opv_ref[...] = jnp.pad(pvt_ref[...], ((0, 16 - CTX), (0, 0)))
    onv_ref[...] = jnp.pad(nvt_ref[...], ((0, 16 - CTX), (0, 0)))


_prep_idx = pl.pallas_call(
    _prep_idx_body,
    out_shape=[
        jax.ShapeDtypeStruct((16, B_POS), jnp.int32),
        jax.ShapeDtypeStruct((16, B_NEG), jnp.int32),
    ],
)


def _sc_body(u_lin, v_lin, pos_u, neg_u, pvt, nvt, out,
             idx_u_all, idx_v_all, rows_u0, rows_u1, rows_v0, rows_v1,
             parts0, parts1, semg0, semg1, semo0, semo1):
    u_emb = u_lin
    v_emb = v_lin
    rows_u = (rows_u0, rows_u1)
    rows_v = (rows_v0, rows_v1)
    parts = (parts0, parts1)
    semg = (semg0, semg1)
    semo = (semo0, semo1)

    wid = lax.axis_index("s") * NC + lax.axis_index("c")
    idx_cps = [
        pltpu.async_copy(pos_u.at[pl.ds(wid * POS_PER_W, POS_PER_W)],
                         idx_u_all.at[pl.ds(0, POS_PER_W)], semg0),
        pltpu.async_copy(neg_u.at[pl.ds(wid * NEG_PER_W, NEG_PER_W)],
                         idx_u_all.at[pl.ds(POS_PER_W, NEG_PER_W)], semg0),
    ]
    for c in range(CTX):
        idx_cps.append(pltpu.async_copy(
            pvt.at[c, pl.ds(wid * POS_PER_W, POS_PER_W)],
            idx_v_all.at[pl.ds(c * POS_PER_W, POS_PER_W)], semg0))
        idx_cps.append(pltpu.async_copy(
            nvt.at[c, pl.ds(wid * NEG_PER_W, NEG_PER_W)],
            idx_v_all.at[pl.ds(POS_CTX_LEN + c * NEG_PER_W, NEG_PER_W)],
            semg0))
    for cp in idx_cps:
        cp.wait()

    def issue(j, b):
        pltpu.async_copy(
            v_emb.at[idx_u_all.at[pl.ds(j * C, C)]], rows_u[b], semg[b])
        jp = jnp.minimum(j, NPOS_CHUNK - 1)
        jn_ = jnp.maximum(j - NPOS_CHUNK, 0)
        is_pos = j < NPOS_CHUNK
        for c in range(CTX):
            off = jnp.where(is_pos, c * POS_PER_W + jp * C,
                            POS_CTX_LEN + c * NEG_PER_W + jn_ * C)
            pltpu.async_copy(
                u_emb.at[idx_v_all.at[pl.ds(off, C)]],
                rows_v[b].at[pl.ds(c * C, C)], semg[b])

    def drain_gathers(b):
        pltpu.make_async_copy(v_emb.at[pl.ds(0, C)], rows_u[b], semg[b]).wait()
        pltpu.make_async_copy(
            u_emb.at[pl.ds(0, C * CTX)], rows_v[b], semg[b]).wait()

    def compute(b):
        rv, ru, pt = rows_v[b], rows_u[b], parts[b]

        def item_body(i, carry):
            accs = [rv[i, pl.ds(d * 16, 16)] for d in range(4)]
            for c in range(1, CTX):
                for d in range(4):
                    accs[d] = accs[d] + rv[c * C + i, pl.ds(d * 16, 16)]
            part = accs[0] * ru[i, pl.ds(0, 16)]
            for d in range(1, 4):
                part = part + accs[d] * ru[i, pl.ds(d * 16, 16)]
            pt[pl.ds(i * 16, 16)] = part
            return carry

        lax.fori_loop(0, C, item_body, 0)

    issue(0, 0)

    def outer(g, carry):
        for b in range(NBUF):
            j = g * NBUF + b
            jn = j + 1

            @pl.when(jn < NCHUNK)
            def _():
                issue(jn, b ^ 1)

            drain_gathers(b)

            @pl.when(j >= NBUF)
            def _():
                pltpu.make_async_copy(
                    parts[b], out.at[pl.ds(0, C * 16)], semo[b]).wait()

            compute(b)
            off = jnp.where(
                j < NPOS_CHUNK,
                wid * POS_PER_W + j * C,
                B_POS + wid * NEG_PER_W + (j - NPOS_CHUNK) * C)
            pltpu.async_copy(
                parts[b], out.at[pl.ds(off * 16, C * 16)], semo[b])
        return carry

    lax.fori_loop(0, NCHUNK // NBUF, outer, 0)
    for b in range(NBUF):
        pltpu.make_async_copy(
            parts[b], out.at[pl.ds(0, C * 16)], semo[b]).wait()


_sc_scores = functools.partial(
    pl.kernel,
    out_type=jax.ShapeDtypeStruct((OUT_LEN,), jnp.float32),
    mesh=plsc.VectorSubcoreMesh(core_axis_name="c", subcore_axis_name="s"),
    scratch_types=[
        pltpu.VMEM((ITEMS_PER_W,), jnp.int32),
        pltpu.VMEM((ITEMS_PER_W * CTX,), jnp.int32),
        pltpu.VMEM((C, EMB_DIM), jnp.float32),
        pltpu.VMEM((C, EMB_DIM), jnp.float32),
        pltpu.VMEM((C * CTX, EMB_DIM), jnp.float32),
        pltpu.VMEM((C * CTX, EMB_DIM), jnp.float32),
        pltpu.VMEM((C * 16,), jnp.float32),
        pltpu.VMEM((C * 16,), jnp.float32),
        pltpu.SemaphoreType.DMA,
        pltpu.SemaphoreType.DMA,
        pltpu.SemaphoreType.DMA,
        pltpu.SemaphoreType.DMA,
    ],
    compiler_params=pltpu.CompilerParams(use_tc_tiling_on_sc=False),
)(_sc_body)


def _tc_body(parts_ref, o_ref):
    x = parts_ref[...]
    sel = (lax.broadcasted_iota(jnp.int32, (128, 8), 0) // 16
           == lax.broadcasted_iota(jnp.int32, (128, 8), 1)).astype(jnp.float32)
    s = jnp.dot(x, sel, preferred_element_type=jnp.float32)
    row = lax.broadcasted_iota(jnp.int32, (R128, 8), 0)
    sign = jnp.where(row < B_POS // 8, 1.0, -1.0)
    t = s * sign
    ls = jnp.minimum(t, 0.0) - jnp.log(1.0 + jnp.exp(-jnp.abs(t)))
    o_ref[0, 0] = -jnp.sum(ls)


_tc_finish = pl.pallas_call(
    _tc_body,
    out_shape=jax.ShapeDtypeStruct((1, 1), jnp.float32),
    out_specs=pl.BlockSpec(memory_space=pltpu.SMEM),
)


def kernel(pos_u, pos_v, neg_u, neg_v, u_emb, v_emb):
    pvt, nvt = _prep_idx(pos_v.astype(jnp.int32).T, neg_v.astype(jnp.int32).T)
    parts = _sc_scores(
        u_emb, v_emb,
        pos_u.astype(jnp.int32), neg_u.astype(jnp.int32), pvt, nvt)
    loss = _tc_finish(parts.reshape(R128, 128))
    return loss[0, 0]
